# baseline (speedup 1.0000x reference)
import functools

import jax
import jax.numpy as jnp
from jax import lax
from jax.experimental import pallas as pl
from jax.experimental.pallas import tpu as pltpu
from jax.experimental.pallas import tpu_sc as plsc

_B = 16384
_D = 100
_DP = 104
_H = 64

_NC = 2
_NS = 16
_NW = _NC * _NS
_BPW = _B // _NW
_CH = 128
_NCH = _BPW // _CH


def _gather_kernel(a_idx_hbm, c_idx_hbm, table_hbm, a_out, c_out,
                   idx_v, rows_v, sem):
    wid = lax.axis_index("s") * _NC + lax.axis_index("c")
    base = wid * _BPW
    for src, dst in ((a_idx_hbm, a_out), (c_idx_hbm, c_out)):
        for ch in range(_NCH):
            off = base + ch * _CH
            pltpu.sync_copy(src.at[pl.ds(off, _CH)], idx_v)
            pltpu.async_copy(table_hbm.at[idx_v], rows_v, sem).wait()
            pltpu.sync_copy(rows_v, dst.at[pl.ds(off, _CH)])


_gather2 = functools.partial(
    pl.kernel,
    mesh=plsc.VectorSubcoreMesh(core_axis_name="c", subcore_axis_name="s"),
    compiler_params=pltpu.CompilerParams(use_tc_tiling_on_sc=False),
    out_type=[jax.ShapeDtypeStruct((_B, _DP), jnp.float32),
              jax.ShapeDtypeStruct((_B, _DP), jnp.float32)],
    scratch_types=[
        pltpu.VMEM((_CH,), jnp.int32),
        pltpu.VMEM((_CH, _DP), jnp.float32),
        pltpu.SemaphoreType.DMA,
    ],
)(_gather_kernel)


_BB = 1024


def _mlp_body(a_ref, c_ref, w1a_ref, w1c_ref, w1m_ref, b1_ref, w2_ref,
              b2_ref, out_ref):
    a = a_ref[...]
    c = c_ref[...]
    h = (jnp.dot(a, w1a_ref[...], preferred_element_type=jnp.float32)
         + jnp.dot(c, w1c_ref[...], preferred_element_type=jnp.float32)
         + jnp.dot(a * c, w1m_ref[...], preferred_element_type=jnp.float32)
         + b1_ref[...])
    h = jnp.maximum(h, 0.0)
    out_ref[...] = (jnp.dot(h, w2_ref[...], preferred_element_type=jnp.float32)
                    + b2_ref[...])


def _mlp(a_rows, c_rows, w1a, w1c, w1m, b1, w2, b2):
    grid = (_B // _BB,)
    return pl.pallas_call(
        _mlp_body,
        grid=grid,
        in_specs=[
            pl.BlockSpec((_BB, _DP), lambda i: (i, 0)),
            pl.BlockSpec((_BB, _DP), lambda i: (i, 0)),
            pl.BlockSpec((_DP, _H), lambda i: (0, 0)),
            pl.BlockSpec((_DP, _H), lambda i: (0, 0)),
            pl.BlockSpec((_DP, _H), lambda i: (0, 0)),
            pl.BlockSpec((1, _H), lambda i: (0, 0)),
            pl.BlockSpec((_H, 1), lambda i: (0, 0)),
            pl.BlockSpec((1, 1), lambda i: (0, 0)),
        ],
        out_specs=pl.BlockSpec((_BB, 1), lambda i: (i, 0)),
        out_shape=jax.ShapeDtypeStruct((_B, 1), jnp.float32),
    )(a_rows, c_rows, w1a, w1c, w1m, b1, w2, b2)


def kernel(anchor_h, candidate_h, doc_embed, W1, b1, W2, b2):
    table = jnp.pad(doc_embed, ((0, 0), (0, _DP - _D)))
    a_rows, c_rows = _gather2(anchor_h, candidate_h, table)
    pad_w = ((0, _DP - _D), (0, 0))
    w1a = jnp.pad(W1[:_D], pad_w)
    w1c = jnp.pad(W1[_D:2 * _D], pad_w)
    w1m = jnp.pad(W1[2 * _D:], pad_w)
    return _mlp(a_rows, c_rows, w1a, w1c, w1m,
                b1.reshape(1, _H), W2, b2.reshape(1, 1))


# speedup vs baseline: 3.0101x; 3.0101x over previous
import functools

import jax
import jax.numpy as jnp
from jax import lax
from jax.experimental import pallas as pl
from jax.experimental.pallas import tpu as pltpu
from jax.experimental.pallas import tpu_sc as plsc

_B = 16384
_D = 100
_DP = 128
_V = 100000
_H = 64


_TBLK = 2048
_TG = (_V + _TBLK - 1) // _TBLK


def _tr_body(t_ref, out_ref):
    x = t_ref[...]
    xt = x.T
    out_ref[...] = jnp.pad(xt, ((0, 0), (0, _DP - _D)))


def _transpose_pad(tableT):
    return pl.pallas_call(
        _tr_body,
        grid=(_TG,),
        in_specs=[pl.BlockSpec((_D, _TBLK), lambda j: (0, j))],
        out_specs=pl.BlockSpec((_TBLK, _DP), lambda j: (j, 0)),
        out_shape=jax.ShapeDtypeStruct((_V, _DP), jnp.float32),
    )(tableT)


_NC = 2
_NS = 16
_NW = _NC * _NS
_BPW = _B // _NW
_CH = 128
_NCH = _BPW // _CH


def _gather_kernel(a_idx_hbm, c_idx_hbm, table_hbm, a_out, c_out,
                   idx_v, rows_v, sem):
    wid = lax.axis_index("s") * _NC + lax.axis_index("c")
    base = wid * _BPW
    for src, dst in ((a_idx_hbm, a_out), (c_idx_hbm, c_out)):
        for ch in range(_NCH):
            off = base + ch * _CH
            pltpu.sync_copy(src.at[pl.ds(off, _CH)], idx_v)
            pltpu.async_copy(table_hbm.at[idx_v], rows_v, sem).wait()
            pltpu.sync_copy(rows_v, dst.at[pl.ds(off, _CH)])


_gather2 = functools.partial(
    pl.kernel,
    mesh=plsc.VectorSubcoreMesh(core_axis_name="c", subcore_axis_name="s"),
    out_type=[jax.ShapeDtypeStruct((_B, _DP), jnp.float32),
              jax.ShapeDtypeStruct((_B, _DP), jnp.float32)],
    scratch_types=[
        pltpu.VMEM((_CH,), jnp.int32),
        pltpu.VMEM((_CH, _DP), jnp.float32),
        pltpu.SemaphoreType.DMA,
    ],
)(_gather_kernel)


_BB = 1024


def _mlp_body(a_ref, c_ref, w1a_ref, w1c_ref, w1m_ref, b1_ref, w2_ref,
              b2_ref, out_ref):
    a = a_ref[...]
    c = c_ref[...]
    h = (jnp.dot(a, w1a_ref[...], preferred_element_type=jnp.float32)
         + jnp.dot(c, w1c_ref[...], preferred_element_type=jnp.float32)
         + jnp.dot(a * c, w1m_ref[...], preferred_element_type=jnp.float32)
         + b1_ref[...])
    h = jnp.maximum(h, 0.0)
    out_ref[...] = (jnp.dot(h, w2_ref[...], preferred_element_type=jnp.float32)
                    + b2_ref[...])


def _mlp(a_rows, c_rows, w1a, w1c, w1m, b1, w2, b2):
    return pl.pallas_call(
        _mlp_body,
        grid=(_B // _BB,),
        in_specs=[
            pl.BlockSpec((_BB, _DP), lambda i: (i, 0)),
            pl.BlockSpec((_BB, _DP), lambda i: (i, 0)),
            pl.BlockSpec((_DP, _H), lambda i: (0, 0)),
            pl.BlockSpec((_DP, _H), lambda i: (0, 0)),
            pl.BlockSpec((_DP, _H), lambda i: (0, 0)),
            pl.BlockSpec((1, _H), lambda i: (0, 0)),
            pl.BlockSpec((_H, 1), lambda i: (0, 0)),
            pl.BlockSpec((1, 1), lambda i: (0, 0)),
        ],
        out_specs=pl.BlockSpec((_BB, 1), lambda i: (i, 0)),
        out_shape=jax.ShapeDtypeStruct((_B, 1), jnp.float32),
    )(a_rows, c_rows, w1a, w1c, w1m, b1, w2, b2)


def kernel(anchor_h, candidate_h, doc_embed, W1, b1, W2, b2):
    table = _transpose_pad(doc_embed.T)
    a_rows, c_rows = _gather2(anchor_h, candidate_h, table)
    pad_w = ((0, _DP - _D), (0, 0))
    w1a = jnp.pad(W1[:_D], pad_w)
    w1c = jnp.pad(W1[_D:2 * _D], pad_w)
    w1m = jnp.pad(W1[2 * _D:], pad_w)
    return _mlp(a_rows, c_rows, w1a, w1c, w1m,
                b1.reshape(1, _H), W2, b2.reshape(1, 1))


# speedup vs baseline: 3.1393x; 1.0429x over previous
import functools

import jax
import jax.numpy as jnp
from jax import lax
from jax.experimental import pallas as pl
from jax.experimental.pallas import tpu as pltpu
from jax.experimental.pallas import tpu_sc as plsc

_B = 16384
_D = 100
_DP = 128
_V = 100000
_H = 64


_TBLK = 2048
_TG = (_V + _TBLK - 1) // _TBLK


def _tr_body(t_ref, out_ref):
    x = t_ref[...]
    xt = x.T
    out_ref[...] = jnp.pad(xt, ((0, 0), (0, _DP - _D)))


def _transpose_pad(tableT):
    return pl.pallas_call(
        _tr_body,
        grid=(_TG,),
        in_specs=[pl.BlockSpec((_D, _TBLK), lambda j: (0, j))],
        out_specs=pl.BlockSpec((_TBLK, _DP), lambda j: (j, 0)),
        out_shape=jax.ShapeDtypeStruct((_V, _DP), jnp.float32),
    )(tableT)


_NC = 2
_NS = 16
_NW = _NC * _NS
_NSPLIT = 2
_BH = _B // _NSPLIT
_BPW = _BH // _NW
_CH = 128
_NCH = _BPW // _CH


def _gather_kernel(a_idx_hbm, c_idx_hbm, table_hbm, a_out, c_out,
                   idx_v, rows_v, sem):
    wid = lax.axis_index("s") * _NC + lax.axis_index("c")
    base = wid * _BPW
    for src, dst in ((a_idx_hbm, a_out), (c_idx_hbm, c_out)):
        for ch in range(_NCH):
            off = base + ch * _CH
            pltpu.sync_copy(src.at[pl.ds(off, _CH)], idx_v)
            pltpu.async_copy(table_hbm.at[idx_v], rows_v, sem).wait()
            pltpu.sync_copy(rows_v, dst.at[pl.ds(off, _CH)])


_gather2 = functools.partial(
    pl.kernel,
    mesh=plsc.VectorSubcoreMesh(core_axis_name="c", subcore_axis_name="s"),
    out_type=[jax.ShapeDtypeStruct((_BH, _DP), jnp.float32),
              jax.ShapeDtypeStruct((_BH, _DP), jnp.float32)],
    scratch_types=[
        pltpu.VMEM((_CH,), jnp.int32),
        pltpu.VMEM((_CH, _DP), jnp.float32),
        pltpu.SemaphoreType.DMA,
    ],
)(_gather_kernel)


_BB = 1024


def _mlp_body(a_ref, c_ref, w1a_ref, w1c_ref, w1m_ref, b1_ref, w2_ref,
              b2_ref, out_ref):
    a = a_ref[...]
    c = c_ref[...]
    h = (jnp.dot(a, w1a_ref[...], preferred_element_type=jnp.float32)
         + jnp.dot(c, w1c_ref[...], preferred_element_type=jnp.float32)
         + jnp.dot(a * c, w1m_ref[...], preferred_element_type=jnp.float32)
         + b1_ref[...])
    h = jnp.maximum(h, 0.0)
    out_ref[...] = (jnp.dot(h, w2_ref[...], preferred_element_type=jnp.float32)
                    + b2_ref[...])


def _mlp(a_rows, c_rows, w1a, w1c, w1m, b1, w2, b2):
    return pl.pallas_call(
        _mlp_body,
        grid=(_BH // _BB,),
        in_specs=[
            pl.BlockSpec((_BB, _DP), lambda i: (i, 0)),
            pl.BlockSpec((_BB, _DP), lambda i: (i, 0)),
            pl.BlockSpec((_DP, _H), lambda i: (0, 0)),
            pl.BlockSpec((_DP, _H), lambda i: (0, 0)),
            pl.BlockSpec((_DP, _H), lambda i: (0, 0)),
            pl.BlockSpec((1, _H), lambda i: (0, 0)),
            pl.BlockSpec((_H, 1), lambda i: (0, 0)),
            pl.BlockSpec((1, 1), lambda i: (0, 0)),
        ],
        out_specs=pl.BlockSpec((_BB, 1), lambda i: (i, 0)),
        out_shape=jax.ShapeDtypeStruct((_BH, 1), jnp.float32),
    )(a_rows, c_rows, w1a, w1c, w1m, b1, w2, b2)


def kernel(anchor_h, candidate_h, doc_embed, W1, b1, W2, b2):
    table = _transpose_pad(doc_embed.T)
    pad_w = ((0, _DP - _D), (0, 0))
    w1a = jnp.pad(W1[:_D], pad_w)
    w1c = jnp.pad(W1[_D:2 * _D], pad_w)
    w1m = jnp.pad(W1[2 * _D:], pad_w)
    b1r = b1.reshape(1, _H)
    b2r = b2.reshape(1, 1)
    gathered = [
        _gather2(anchor_h[k * _BH:(k + 1) * _BH],
                 candidate_h[k * _BH:(k + 1) * _BH], table)
        for k in range(_NSPLIT)
    ]
    outs = [_mlp(a_rows, c_rows, w1a, w1c, w1m, b1r, W2, b2r)
            for (a_rows, c_rows) in gathered]
    return jnp.concatenate(outs, axis=0)


# speedup vs baseline: 3.1411x; 1.0006x over previous
import functools

import jax
import jax.numpy as jnp
from jax import lax
from jax.experimental import pallas as pl
from jax.experimental.pallas import tpu as pltpu
from jax.experimental.pallas import tpu_sc as plsc

_B = 16384
_D = 100
_DP = 128
_W = _DP // 2
_V = 100000
_H = 64


_TBLK = 2048
_TG = (_V + _TBLK - 1) // _TBLK
_VP = _TG * _TBLK
_HBLK = _TBLK // 2


def _tr_body(t_ref, out_ref):
    x = t_ref[...]
    xt = jnp.pad(x.T, ((0, 0), (0, _DP - _D)))
    u = lax.bitcast_convert_type(xt, jnp.uint32)
    b = (u + 0x7FFF + ((u >> 16) & 1)) >> 16
    w = (b[:, :_W] | (b[:, _W:] << 16)).astype(jnp.int32)
    out_ref[...] = jnp.concatenate([w[:_HBLK, :], w[_HBLK:, :]], axis=1)


def _transpose_pack(tableT):
    return pl.pallas_call(
        _tr_body,
        grid=(_TG,),
        in_specs=[pl.BlockSpec((_D, _TBLK), lambda j: (0, j))],
        out_specs=pl.BlockSpec((_HBLK, _DP), lambda j: (j, 0)),
        out_shape=jax.ShapeDtypeStruct((_VP // 2, _DP), jnp.int32),
    )(tableT)


_NC = 2
_NS = 16
_NW = _NC * _NS
_NSPLIT = 2
_BH = _B // _NSPLIT
_BPW = _BH // _NW
_CH = 128
_NCH = _BPW // _CH
_L = 16


def _sigma(t):
    return ((t >> 11) << 11) + ((t & 1023) << 1) + ((t >> 10) & 1)


def _gather_kernel(a_idx_hbm, c_idx_hbm, table_hbm, a_out, c_out,
                   idx_v, sidx_v, rows_v, sem):
    wid = lax.axis_index("s") * _NC + lax.axis_index("c")
    base = wid * _BPW
    for src, dst in ((a_idx_hbm, a_out), (c_idx_hbm, c_out)):
        for ch in range(_NCH):
            off = base + ch * _CH
            pltpu.sync_copy(src.at[pl.ds(off, _CH)], idx_v)
            for k in range(_CH // _L):
                t = idx_v[pl.ds(k * _L, _L)]
                sidx_v[pl.ds(k * _L, _L)] = _sigma(t)
            pltpu.async_copy(table_hbm.at[sidx_v], rows_v, sem).wait()
            pltpu.sync_copy(rows_v, dst.at[pl.ds(off, _CH)])


def _make_gather2():
    return functools.partial(
        pl.kernel,
        mesh=plsc.VectorSubcoreMesh(core_axis_name="c", subcore_axis_name="s"),
        compiler_params=pltpu.CompilerParams(use_tc_tiling_on_sc=False),
        out_type=[jax.ShapeDtypeStruct((_BH, _W), jnp.int32),
                  jax.ShapeDtypeStruct((_BH, _W), jnp.int32)],
        scratch_types=[
            pltpu.VMEM((_CH,), jnp.int32),
            pltpu.VMEM((_CH,), jnp.int32),
            pltpu.VMEM((_CH, _W), jnp.int32),
            pltpu.SemaphoreType.DMA,
        ],
    )(_gather_kernel)


_BB = 512


def _unpack(wm):
    lo = lax.bitcast_convert_type(wm << 16, jnp.float32)
    hi = lax.bitcast_convert_type(wm & -65536, jnp.float32)
    return lo, hi


def _mlp_body(a_ref, c_ref, wal_ref, wah_ref, wcl_ref, wch_ref, wml_ref,
              wmh_ref, b1_ref, w2_ref, b2_ref, out_ref):
    wa = a_ref[...]
    wc = c_ref[...]
    a_lo, a_hi = _unpack(wa)
    c_lo, c_hi = _unpack(wc)

    def head(al, ah, cl, chh):
        h = (jnp.dot(al, wal_ref[...], preferred_element_type=jnp.float32)
             + jnp.dot(ah, wah_ref[...], preferred_element_type=jnp.float32)
             + jnp.dot(cl, wcl_ref[...], preferred_element_type=jnp.float32)
             + jnp.dot(chh, wch_ref[...], preferred_element_type=jnp.float32)
             + jnp.dot(al * cl, wml_ref[...],
                       preferred_element_type=jnp.float32)
             + jnp.dot(ah * chh, wmh_ref[...],
                       preferred_element_type=jnp.float32)
             + b1_ref[...])
        h = jnp.maximum(h, 0.0)
        return (jnp.dot(h, w2_ref[...], preferred_element_type=jnp.float32)
                + b2_ref[...])

    oe = head(a_lo[:, :_H], a_hi[:, :_H], c_lo[:, :_H], c_hi[:, :_H])
    oo = head(a_lo[:, _H:], a_hi[:, _H:], c_lo[:, _H:], c_hi[:, _H:])
    out_ref[...] = jnp.concatenate([oe, oo], axis=1)


def _mlp(a_pk, c_pk, ws, b1, w2, b2):
    full = lambda i: (0, 0)
    return pl.pallas_call(
        _mlp_body,
        grid=(_BH // 2 // _BB,),
        in_specs=[
            pl.BlockSpec((_BB, _DP), lambda i: (i, 0)),
            pl.BlockSpec((_BB, _DP), lambda i: (i, 0)),
        ] + [pl.BlockSpec((_H, _H), full) for _ in range(6)] + [
            pl.BlockSpec((1, _H), full),
            pl.BlockSpec((_H, 1), full),
            pl.BlockSpec((1, 1), full),
        ],
        out_specs=pl.BlockSpec((_BB, 2), lambda i: (i, 0)),
        out_shape=jax.ShapeDtypeStruct((_BH // 2, 2), jnp.float32),
    )(a_pk, c_pk, *ws, b1, w2, b2)


def kernel(anchor_h, candidate_h, doc_embed, W1, b1, W2, b2):
    table = _transpose_pack(doc_embed.T)
    table = table.reshape(_VP, _W)
    w1a = jnp.pad(W1[:_D], ((0, _DP - _D), (0, 0)))
    w1c = jnp.pad(W1[_D:2 * _D], ((0, _DP - _D), (0, 0)))
    w1m = jnp.pad(W1[2 * _D:], ((0, _DP - _D), (0, 0)))
    ws = (w1a[:_H], w1a[_H:], w1c[:_H], w1c[_H:], w1m[:_H], w1m[_H:])
    b1r = b1.reshape(1, _H)
    b2r = b2.reshape(1, 1)
    _gather2 = _make_gather2()
    gathered = [
        _gather2(anchor_h[k * _BH:(k + 1) * _BH],
                 candidate_h[k * _BH:(k + 1) * _BH], table)
        for k in range(_NSPLIT)
    ]
    outs = [
        _mlp(a_pk.reshape(_BH // 2, _DP), c_pk.reshape(_BH // 2, _DP),
             ws, b1r, W2, b2r).reshape(_BH, 1)
        for (a_pk, c_pk) in gathered
    ]
    return jnp.concatenate(outs, axis=0)


# speedup vs baseline: 3.9280x; 1.2505x over previous
import functools

import jax
import jax.numpy as jnp
from jax import lax
from jax.experimental import pallas as pl
from jax.experimental.pallas import tpu as pltpu
from jax.experimental.pallas import tpu_sc as plsc

_B = 16384
_D = 100
_DP = 128
_W = _DP // 2
_V = 100000
_H = 64


_TBLK = 8192
_TG = (_V + _TBLK - 1) // _TBLK
_VP = _TG * _TBLK
_HBLK = _TBLK // 2


def _tr_body(t_ref, out_ref):
    x = t_ref[...]
    xt = jnp.pad(x.T, ((0, 0), (0, _DP - _D)))
    u = lax.bitcast_convert_type(xt, jnp.uint32)
    b = (u + 0x7FFF + ((u >> 16) & 1)) >> 16
    w = (b[:, :_W] | (b[:, _W:] << 16)).astype(jnp.int32)
    out_ref[...] = jnp.concatenate([w[:_HBLK, :], w[_HBLK:, :]], axis=1)


def _transpose_pack(tableT):
    return pl.pallas_call(
        _tr_body,
        grid=(_TG,),
        in_specs=[pl.BlockSpec((_D, _TBLK), lambda j: (0, j))],
        out_specs=pl.BlockSpec((_HBLK, _DP), lambda j: (j, 0)),
        out_shape=jax.ShapeDtypeStruct((_VP // 2, _DP), jnp.int32),
    )(tableT)


_NC = 2
_NS = 16
_NW = _NC * _NS
_NSPLIT = 2
_BH = _B // _NSPLIT
_BPW = _BH // _NW
_CH = 128
_NCH = _BPW // _CH
_L = 16


_SH = _TBLK.bit_length() - 1
_HM = _HBLK - 1


def _sigma(t):
    return ((t >> _SH) << _SH) + ((t & _HM) << 1) + ((t >> (_SH - 1)) & 1)


def _gather_kernel(half, a_idx_hbm, c_idx_hbm, table_hbm, a_out, c_out,
                   idx_v, sidx_v, rows_v, sem):
    wid = lax.axis_index("s") * _NC + lax.axis_index("c")
    base = half * _BH + wid * _BPW
    obase = wid * _BPW
    for src, dst in ((a_idx_hbm, a_out), (c_idx_hbm, c_out)):
        for ch in range(_NCH):
            off = base + ch * _CH
            pltpu.sync_copy(src.at[pl.ds(off, _CH)], idx_v)
            for k in range(_CH // _L):
                t = idx_v[pl.ds(k * _L, _L)]
                sidx_v[pl.ds(k * _L, _L)] = _sigma(t)
            pltpu.async_copy(table_hbm.at[sidx_v], rows_v, sem).wait()
            pltpu.sync_copy(rows_v, dst.at[pl.ds(obase + ch * _CH, _CH)])


def _make_gather2(half):
    return functools.partial(
        pl.kernel,
        mesh=plsc.VectorSubcoreMesh(core_axis_name="c", subcore_axis_name="s"),
        compiler_params=pltpu.CompilerParams(use_tc_tiling_on_sc=False),
        out_type=[jax.ShapeDtypeStruct((_BH, _W), jnp.int32),
                  jax.ShapeDtypeStruct((_BH, _W), jnp.int32)],
        scratch_types=[
            pltpu.VMEM((_CH,), jnp.int32),
            pltpu.VMEM((_CH,), jnp.int32),
            pltpu.VMEM((_CH, _W), jnp.int32),
            pltpu.SemaphoreType.DMA,
        ],
    )(functools.partial(_gather_kernel, half))


_BB = 1024


def _unpack(wm):
    lo = lax.bitcast_convert_type(wm << 16, jnp.float32)
    hi = lax.bitcast_convert_type(wm & -65536, jnp.float32)
    return lo, hi


def _mlp_body(a_ref, c_ref, wal_ref, wah_ref, wcl_ref, wch_ref, wml_ref,
              wmh_ref, b1_ref, w2_ref, b2_ref, out_ref):
    wa = a_ref[...]
    wc = c_ref[...]
    a_lo, a_hi = _unpack(wa)
    c_lo, c_hi = _unpack(wc)

    def head(al, ah, cl, chh):
        h = (jnp.dot(al, wal_ref[...], preferred_element_type=jnp.float32)
             + jnp.dot(ah, wah_ref[...], preferred_element_type=jnp.float32)
             + jnp.dot(cl, wcl_ref[...], preferred_element_type=jnp.float32)
             + jnp.dot(chh, wch_ref[...], preferred_element_type=jnp.float32)
             + jnp.dot(al * cl, wml_ref[...],
                       preferred_element_type=jnp.float32)
             + jnp.dot(ah * chh, wmh_ref[...],
                       preferred_element_type=jnp.float32)
             + b1_ref[...])
        h = jnp.maximum(h, 0.0)
        return (jnp.dot(h, w2_ref[...], preferred_element_type=jnp.float32)
                + b2_ref[...])

    oe = head(a_lo[:, :_H], a_hi[:, :_H], c_lo[:, :_H], c_hi[:, :_H])
    oo = head(a_lo[:, _H:], a_hi[:, _H:], c_lo[:, _H:], c_hi[:, _H:])
    out_ref[...] = jnp.concatenate([oe, oo], axis=1)


def _mlp(a_pk, c_pk, ws, b1, w2, b2):
    full = lambda i: (0, 0)
    return pl.pallas_call(
        _mlp_body,
        grid=(_BH // 2 // _BB,),
        in_specs=[
            pl.BlockSpec((_BB, _DP), lambda i: (i, 0)),
            pl.BlockSpec((_BB, _DP), lambda i: (i, 0)),
        ] + [pl.BlockSpec((_H, _H), full) for _ in range(6)] + [
            pl.BlockSpec((1, _H), full),
            pl.BlockSpec((_H, 1), full),
            pl.BlockSpec((1, 1), full),
        ],
        out_specs=pl.BlockSpec((_BB, 2), lambda i: (i, 0)),
        out_shape=jax.ShapeDtypeStruct((_BH // 2, 2), jnp.float32),
    )(a_pk, c_pk, *ws, b1, w2, b2)


def kernel(anchor_h, candidate_h, doc_embed, W1, b1, W2, b2):
    table = _transpose_pack(doc_embed.T)
    table = table.reshape(_VP, _W)
    w1a = jnp.pad(W1[:_D], ((0, _DP - _D), (0, 0)))
    w1c = jnp.pad(W1[_D:2 * _D], ((0, _DP - _D), (0, 0)))
    w1m = jnp.pad(W1[2 * _D:], ((0, _DP - _D), (0, 0)))
    ws = (w1a[:_H], w1a[_H:], w1c[:_H], w1c[_H:], w1m[:_H], w1m[_H:])
    b1r = b1.reshape(1, _H)
    b2r = b2.reshape(1, 1)
    gathered = [
        _make_gather2(k)(anchor_h, candidate_h, table)
        for k in range(_NSPLIT)
    ]
    outs = [
        _mlp(a_pk.reshape(_BH // 2, _DP), c_pk.reshape(_BH // 2, _DP),
             ws, b1r, W2, b2r).reshape(_BH, 1)
        for (a_pk, c_pk) in gathered
    ]
    return jnp.concatenate(outs, axis=0)


# speedup vs baseline: 4.0970x; 1.0430x over previous
import functools

import jax
import jax.numpy as jnp
from jax import lax
from jax.experimental import pallas as pl
from jax.experimental.pallas import tpu as pltpu
from jax.experimental.pallas import tpu_sc as plsc

_B = 16384
_D = 100
_DP = 128
_W = _DP // 2
_V = 100000
_H = 64


_TBLK = 16384
_TG = (_V + _TBLK - 1) // _TBLK
_VP = _TG * _TBLK
_HBLK = _TBLK // 2


def _tr_body(t_ref, out_ref):
    x = t_ref[...]
    xt = jnp.pad(x.T, ((0, 0), (0, _DP - _D)))
    u = lax.bitcast_convert_type(xt, jnp.uint32)
    b = (u + 0x7FFF + ((u >> 16) & 1)) >> 16
    w = (b[:, :_W] | (b[:, _W:] << 16)).astype(jnp.int32)
    out_ref[...] = jnp.concatenate([w[:_HBLK, :], w[_HBLK:, :]], axis=1)


def _transpose_pack(tableT):
    return pl.pallas_call(
        _tr_body,
        grid=(_TG,),
        in_specs=[pl.BlockSpec((_D, _TBLK), lambda j: (0, j))],
        out_specs=pl.BlockSpec((_HBLK, _DP), lambda j: (j, 0)),
        out_shape=jax.ShapeDtypeStruct((_VP // 2, _DP), jnp.int32),
    )(tableT)


_NC = 2
_NS = 16
_NW = _NC * _NS
_NSPLIT = 2
_BH = _B // _NSPLIT
_BPW = _BH // _NW
_CH = 128
_NCH = _BPW // _CH
_L = 16


_SH = _TBLK.bit_length() - 1
_HM = _HBLK - 1


def _sigma(t):
    return ((t >> _SH) << _SH) + ((t & _HM) << 1) + ((t >> (_SH - 1)) & 1)


def _gather_kernel(half, a_idx_hbm, c_idx_hbm, table_hbm, a_out, c_out,
                   idx_v, sidx_v, rows_v, sem):
    wid = lax.axis_index("s") * _NC + lax.axis_index("c")
    base = half * _BH + wid * _BPW
    obase = wid * _BPW
    for src, dst in ((a_idx_hbm, a_out), (c_idx_hbm, c_out)):
        for ch in range(_NCH):
            off = base + ch * _CH
            pltpu.sync_copy(src.at[pl.ds(off, _CH)], idx_v)
            for k in range(_CH // _L):
                t = idx_v[pl.ds(k * _L, _L)]
                sidx_v[pl.ds(k * _L, _L)] = _sigma(t)
            pltpu.async_copy(table_hbm.at[sidx_v], rows_v, sem).wait()
            pltpu.sync_copy(rows_v, dst.at[pl.ds(obase + ch * _CH, _CH)])


def _make_gather2(half):
    return functools.partial(
        pl.kernel,
        mesh=plsc.VectorSubcoreMesh(core_axis_name="c", subcore_axis_name="s"),
        compiler_params=pltpu.CompilerParams(use_tc_tiling_on_sc=False),
        out_type=[jax.ShapeDtypeStruct((_BH, _W), jnp.int32),
                  jax.ShapeDtypeStruct((_BH, _W), jnp.int32)],
        scratch_types=[
            pltpu.VMEM((_CH,), jnp.int32),
            pltpu.VMEM((_CH,), jnp.int32),
            pltpu.VMEM((_CH, _W), jnp.int32),
            pltpu.SemaphoreType.DMA,
        ],
    )(functools.partial(_gather_kernel, half))


_BB = 1024


def _unpack(wm):
    lo = lax.bitcast_convert_type(wm << 16, jnp.float32)
    hi = lax.bitcast_convert_type(wm & -65536, jnp.float32)
    return lo, hi


def _mlp_body(a_ref, c_ref, ws_ref, b1_ref, w2_ref, b2_ref, out_ref):
    wa = a_ref[...]
    wc = c_ref[...]
    a_lo, a_hi = _unpack(wa)
    c_lo, c_hi = _unpack(wc)

    def feats(al, ah, cl, chh):
        return jnp.concatenate([al, ah, cl, chh, al * cl, ah * chh], axis=1)

    xe = feats(a_lo[:, :_H], a_hi[:, :_H], c_lo[:, :_H], c_hi[:, :_H])
    xo = feats(a_lo[:, _H:], a_hi[:, _H:], c_lo[:, _H:], c_hi[:, _H:])
    x = jnp.concatenate([xe, xo], axis=0).astype(jnp.bfloat16)
    h = (jnp.dot(x, ws_ref[...], preferred_element_type=jnp.float32)
         + b1_ref[...])
    h = jnp.maximum(h, 0.0)
    o = (lax.dot_general(w2_ref[...], h, (((0,), (1,)), ((), ())),
                         preferred_element_type=jnp.float32)
         + b2_ref[...])
    out_ref[0:1, :] = o[:, :_BB]
    out_ref[1:2, :] = o[:, _BB:]


def _mlp(a_pk, c_pk, ws, b1, w2, b2):
    full = lambda i: (0, 0)
    return pl.pallas_call(
        _mlp_body,
        grid=(_BH // 2 // _BB,),
        in_specs=[
            pl.BlockSpec((_BB, _DP), lambda i: (i, 0)),
            pl.BlockSpec((_BB, _DP), lambda i: (i, 0)),
            pl.BlockSpec((6 * _H, _H), full),
            pl.BlockSpec((1, _H), full),
            pl.BlockSpec((_H, 1), full),
            pl.BlockSpec((1, 1), full),
        ],
        out_specs=pl.BlockSpec((2, _BB), lambda i: (0, i)),
        out_shape=jax.ShapeDtypeStruct((2, _BH // 2), jnp.float32),
    )(a_pk, c_pk, ws, b1, w2, b2)


def kernel(anchor_h, candidate_h, doc_embed, W1, b1, W2, b2):
    table = _transpose_pack(doc_embed.T)
    table = table.reshape(_VP, _W)
    w1a = jnp.pad(W1[:_D], ((0, _DP - _D), (0, 0)))
    w1c = jnp.pad(W1[_D:2 * _D], ((0, _DP - _D), (0, 0)))
    w1m = jnp.pad(W1[2 * _D:], ((0, _DP - _D), (0, 0)))
    ws = jnp.concatenate(
        [w1a[:_H], w1a[_H:], w1c[:_H], w1c[_H:], w1m[:_H], w1m[_H:]],
        axis=0).astype(jnp.bfloat16)
    b1r = b1.reshape(1, _H)
    b2r = b2.reshape(1, 1)
    gathered = [
        _make_gather2(k)(anchor_h, candidate_h, table)
        for k in range(_NSPLIT)
    ]
    outs = [
        _mlp(a_pk.reshape(_BH // 2, _DP), c_pk.reshape(_BH // 2, _DP),
             ws, b1r, W2, b2r)
        for (a_pk, c_pk) in gathered
    ]
    full = jnp.concatenate(outs, axis=1)
    return jnp.transpose(full).reshape(_B, 1)


# speedup vs baseline: 4.4490x; 1.0859x over previous
import functools

import jax
import jax.numpy as jnp
from jax import lax
from jax.experimental import pallas as pl
from jax.experimental.pallas import tpu as pltpu
from jax.experimental.pallas import tpu_sc as plsc

_B = 16384
_D = 100
_DP = 128
_W = _DP // 2
_V = 100000
_H = 64


_TBLK = 8192
_TG = (_V + _TBLK - 1) // _TBLK
_VP = _TG * _TBLK
_HBLK = _TBLK // 2


def _tr_body(t_ref, out_ref):
    x = t_ref[...]
    xt = jnp.pad(x.T, ((0, 0), (0, _DP - _D)))
    u = lax.bitcast_convert_type(xt, jnp.uint32)
    b = (u + 0x7FFF + ((u >> 16) & 1)) >> 16
    w = (b[:, :_W] | (b[:, _W:] << 16)).astype(jnp.int32)
    out_ref[...] = jnp.concatenate([w[:_HBLK, :], w[_HBLK:, :]], axis=1)


def _transpose_pack(tableT):
    return pl.pallas_call(
        _tr_body,
        grid=(_TG,),
        in_specs=[pl.BlockSpec((_D, _TBLK), lambda j: (0, j))],
        out_specs=pl.BlockSpec((_HBLK, _DP), lambda j: (j, 0)),
        out_shape=jax.ShapeDtypeStruct((_VP // 2, _DP), jnp.int32),
    )(tableT)


_NC = 2
_NS = 16
_NW = _NC * _NS
_NSPLIT = 2
_BH = _B // _NSPLIT
_BPW = _BH // _NW
_CH = 128
_NCH = _BPW // _CH
_L = 16


_SH = _TBLK.bit_length() - 1
_HM = _HBLK - 1


def _sigma(t):
    return ((t >> _SH) << _SH) + ((t & _HM) << 1) + ((t >> (_SH - 1)) & 1)


def _tau(b):
    return ((b >> 11) << 11) + ((b & 1023) << 1) + ((b >> 10) & 1)


def _gather_kernel(half, a_idx_hbm, c_idx_hbm, table_hbm, a_out, c_out,
                   idx_v, sidx_v, oidx_v, rows_v0, rows_v1, sem_g, sem_s):
    wid = lax.axis_index("s") * _NC + lax.axis_index("c")
    base = half * _BH + wid * _BPW
    obase = wid * _BPW
    rows_bufs = (rows_v0, rows_v1)
    tasks = [(src, dst, ch)
             for src, dst in ((a_idx_hbm, a_out), (c_idx_hbm, c_out))
             for ch in range(_NCH)]
    iota = lax.iota(jnp.int32, _L)
    scatters = []
    for i, (src, dst, ch) in enumerate(tasks):
        rows_v = rows_bufs[i % 2]
        off = base + ch * _CH
        pltpu.sync_copy(src.at[pl.ds(off, _CH)], idx_v)
        ob = obase + ch * _CH
        for k in range(_CH // _L):
            t = idx_v[pl.ds(k * _L, _L)]
            sidx_v[pl.ds(k * _L, _L)] = _sigma(t)
            oidx_v[pl.ds(k * _L, _L)] = _tau(ob + k * _L + iota)
        if i >= 2:
            scatters[i - 2].wait()
        pltpu.async_copy(table_hbm.at[sidx_v], rows_v, sem_g).wait()
        scatters.append(
            pltpu.async_copy(rows_v, dst.at[oidx_v], sem_s))
    for h in scatters[-2:]:
        h.wait()


def _make_gather2(half):
    return functools.partial(
        pl.kernel,
        mesh=plsc.VectorSubcoreMesh(core_axis_name="c", subcore_axis_name="s"),
        compiler_params=pltpu.CompilerParams(use_tc_tiling_on_sc=False),
        out_type=[jax.ShapeDtypeStruct((_BH, _W), jnp.int32),
                  jax.ShapeDtypeStruct((_BH, _W), jnp.int32)],
        scratch_types=[
            pltpu.VMEM((_CH,), jnp.int32),
            pltpu.VMEM((_CH,), jnp.int32),
            pltpu.VMEM((_CH,), jnp.int32),
            pltpu.VMEM((_CH, _W), jnp.int32),
            pltpu.VMEM((_CH, _W), jnp.int32),
            pltpu.SemaphoreType.DMA,
            pltpu.SemaphoreType.DMA,
        ],
    )(functools.partial(_gather_kernel, half))


_BB = 1024


def _unpack(wm):
    lo = lax.bitcast_convert_type(wm << 16, jnp.float32)
    hi = lax.bitcast_convert_type(wm & -65536, jnp.float32)
    return lo, hi


def _mlp_body(a_ref, c_ref, ws_ref, b1_ref, w2_ref, b2_ref, out_ref):
    wa = a_ref[...]
    wc = c_ref[...]
    a_lo, a_hi = _unpack(wa)
    c_lo, c_hi = _unpack(wc)

    def feats(al, ah, cl, chh):
        return jnp.concatenate([al, ah, cl, chh, al * cl, ah * chh], axis=1)

    xe = feats(a_lo[:, :_H], a_hi[:, :_H], c_lo[:, :_H], c_hi[:, :_H])
    xo = feats(a_lo[:, _H:], a_hi[:, _H:], c_lo[:, _H:], c_hi[:, _H:])
    x = jnp.concatenate([xe, xo], axis=0).astype(jnp.bfloat16)
    h = (jnp.dot(x, ws_ref[...], preferred_element_type=jnp.float32)
         + b1_ref[...])
    h = jnp.maximum(h, 0.0)
    o = (lax.dot_general(w2_ref[...], h, (((0,), (1,)), ((), ())),
                         preferred_element_type=jnp.float32)
         + b2_ref[...])
    out_ref[...] = o


def _mlp(a_pk, c_pk, ws, b1, w2, b2):
    full = lambda i: (0, 0)
    return pl.pallas_call(
        _mlp_body,
        grid=(_BH // 2 // _BB,),
        in_specs=[
            pl.BlockSpec((_BB, _DP), lambda i: (i, 0)),
            pl.BlockSpec((_BB, _DP), lambda i: (i, 0)),
            pl.BlockSpec((6 * _H, _H), full),
            pl.BlockSpec((1, _H), full),
            pl.BlockSpec((_H, 1), full),
            pl.BlockSpec((1, 1), full),
        ],
        out_specs=pl.BlockSpec((1, 2 * _BB), lambda i: (0, i)),
        out_shape=jax.ShapeDtypeStruct((1, _BH), jnp.float32),
    )(a_pk, c_pk, ws, b1, w2, b2)


def kernel(anchor_h, candidate_h, doc_embed, W1, b1, W2, b2):
    table = _transpose_pack(doc_embed.T)
    table = table.reshape(_VP, _W)
    w1a = jnp.pad(W1[:_D], ((0, _DP - _D), (0, 0)))
    w1c = jnp.pad(W1[_D:2 * _D], ((0, _DP - _D), (0, 0)))
    w1m = jnp.pad(W1[2 * _D:], ((0, _DP - _D), (0, 0)))
    ws = jnp.concatenate(
        [w1a[:_H], w1a[_H:], w1c[:_H], w1c[_H:], w1m[:_H], w1m[_H:]],
        axis=0).astype(jnp.bfloat16)
    b1r = b1.reshape(1, _H)
    b2r = b2.reshape(1, 1)
    gathered = [
        _make_gather2(k)(anchor_h, candidate_h, table)
        for k in range(_NSPLIT)
    ]
    outs = [
        _mlp(a_pk.reshape(_BH // 2, _DP), c_pk.reshape(_BH // 2, _DP),
             ws, b1r, W2, b2r)
        for (a_pk, c_pk) in gathered
    ]
    full = jnp.concatenate(outs, axis=1)
    return full.reshape(_B, 1)
